# Initial kernel scaffold; baseline (speedup 1.0000x reference)
import jax
import jax.numpy as jnp
from jax.experimental import pallas as pl


def kernel(node_features, edge_index, W_emb, b_emb, W_g0, b_g0, W_g1, b_g1, W_g2, b_g2, W_p1, b_p1, W_p2, b_p2):
    raise NotImplementedError("write your pallas kernel here")


# speedup vs baseline: 3.4273x; 3.4273x over previous
import functools

import jax
import jax.numpy as jnp
from jax import lax
from jax.experimental import pallas as pl
from jax.experimental.pallas import tpu as pltpu
from jax.experimental.pallas import tpu_sc as plsc

_N, _D, _H = 10000, 128, 256
_HH = _H // 2
_R = 1000
_C = 80
_NC, _NS = 2, 16
_NP = 10240

_INTERPRET = False


def _embed_body(x_ref, wemb_ref, bemb_ref, w_ref, b_ref, h_ref, a_ref, bb_ref):
    x = x_ref[...]
    h = jnp.maximum(
        jnp.dot(x, wemb_ref[...], preferred_element_type=jnp.float32)
        + bemb_ref[...], 0.0)
    h_ref[...] = h
    w = w_ref[...]
    a = jnp.dot(h, w[:_H], preferred_element_type=jnp.float32)
    bb = jnp.dot(h, w[_H:], preferred_element_type=jnp.float32) + b_ref[...]
    a_ref[0] = a[:, :_HH]
    a_ref[1] = a[:, _HH:]
    bb_ref[0] = bb[:, :_HH]
    bb_ref[1] = bb[:, _HH:]


def _combine_body(h_ref, m_ref, w_ref, b_ref, hn_ref, a_ref, bb_ref):
    hn = h_ref[...] + jnp.concatenate([m_ref[0], m_ref[1]], axis=1)
    hn_ref[...] = hn
    w = w_ref[...]
    a = jnp.dot(hn, w[:_H], preferred_element_type=jnp.float32)
    bb = jnp.dot(hn, w[_H:], preferred_element_type=jnp.float32) + b_ref[...]
    a_ref[0] = a[:, :_HH]
    a_ref[1] = a[:, _HH:]
    bb_ref[0] = bb[:, :_HH]
    bb_ref[1] = bb[:, _HH:]


def _final_body(h_ref, m_ref, wp1_ref, bp1_ref, wp2_ref, bp2_ref,
                hf_ref, pred_ref):
    hn = h_ref[...] + jnp.concatenate([m_ref[0], m_ref[1]], axis=1)
    hf_ref[...] = hn
    mean = jnp.mean(hn, axis=0, keepdims=True)
    pooled = jnp.concatenate([hn[0:1, :], mean], axis=1)
    p = jnp.maximum(
        jnp.dot(pooled, wp1_ref[...], preferred_element_type=jnp.float32)
        + bp1_ref[...], 0.0)
    pred_ref[...] = (
        jnp.dot(p, wp2_ref[...], preferred_element_type=jnp.float32)
        + bp2_ref[...])


def _mm_embed(x, wemb, bemb, w, b):
    return pl.pallas_call(
        _embed_body,
        grid=(_N // _R,),
        in_specs=[
            pl.BlockSpec((_R, _D), lambda i: (i, 0)),
            pl.BlockSpec((_D, _H), lambda i: (0, 0)),
            pl.BlockSpec((1, _H), lambda i: (0, 0)),
            pl.BlockSpec((2 * _H, _H), lambda i: (0, 0)),
            pl.BlockSpec((1, _H), lambda i: (0, 0)),
        ],
        out_specs=[
            pl.BlockSpec((_R, _H), lambda i: (i, 0)),
            pl.BlockSpec((2, _R, _HH), lambda i: (0, i, 0)),
            pl.BlockSpec((2, _R, _HH), lambda i: (0, i, 0)),
        ],
        out_shape=[
            jax.ShapeDtypeStruct((_N, _H), jnp.float32),
            jax.ShapeDtypeStruct((2, _N, _HH), jnp.float32),
            jax.ShapeDtypeStruct((2, _N, _HH), jnp.float32),
        ],
        interpret=_INTERPRET,
    )(x, wemb, bemb, w, b)


def _mm_combine(h, m, w, b):
    return pl.pallas_call(
        _combine_body,
        grid=(_N // _R,),
        in_specs=[
            pl.BlockSpec((_R, _H), lambda i: (i, 0)),
            pl.BlockSpec((2, _R, _HH), lambda i: (0, i, 0)),
            pl.BlockSpec((2 * _H, _H), lambda i: (0, 0)),
            pl.BlockSpec((1, _H), lambda i: (0, 0)),
        ],
        out_specs=[
            pl.BlockSpec((_R, _H), lambda i: (i, 0)),
            pl.BlockSpec((2, _R, _HH), lambda i: (0, i, 0)),
            pl.BlockSpec((2, _R, _HH), lambda i: (0, i, 0)),
        ],
        out_shape=[
            jax.ShapeDtypeStruct((_N, _H), jnp.float32),
            jax.ShapeDtypeStruct((2, _N, _HH), jnp.float32),
            jax.ShapeDtypeStruct((2, _N, _HH), jnp.float32),
        ],
        interpret=_INTERPRET,
    )(h, m, w, b)


def _mm_final(h, m, wp1, bp1, wp2, bp2):
    return pl.pallas_call(
        _final_body,
        grid=(1,),
        in_specs=[
            pl.BlockSpec((_N, _H), lambda i: (0, 0)),
            pl.BlockSpec((2, _N, _HH), lambda i: (0, 0, 0)),
            pl.BlockSpec((2 * _H, _H), lambda i: (0, 0)),
            pl.BlockSpec((1, _H), lambda i: (0, 0)),
            pl.BlockSpec((_H, _HH), lambda i: (0, 0)),
            pl.BlockSpec((1, _HH), lambda i: (0, 0)),
        ],
        out_specs=[
            pl.BlockSpec((_N, _H), lambda i: (0, 0)),
            pl.BlockSpec((1, _HH), lambda i: (0, 0)),
        ],
        out_shape=[
            jax.ShapeDtypeStruct((_N, _H), jnp.float32),
            jax.ShapeDtypeStruct((1, _HH), jnp.float32),
        ],
        interpret=_INTERPRET,
    )(h, m, wp1, bp1, wp2, bp2)


def _make_sc_layer(E):
    ept = E // _NS
    chunks = ept // _C
    rpt = _NP // _NS
    mesh = plsc.VectorSubcoreMesh(core_axis_name="c", subcore_axis_name="s",
                                  num_cores=_NC, num_subcores=_NS)

    @functools.partial(
        pl.kernel,
        out_type=jax.ShapeDtypeStruct((_NC * _NP, _HH), jnp.float32),
        mesh=mesh,
        scratch_types=[
            pltpu.VMEM_SHARED((_NP, _HH), jnp.float32),
            pltpu.VMEM((_C,), jnp.int32),
            pltpu.VMEM((_C,), jnp.int32),
            pltpu.VMEM((_C,), jnp.int32),
            pltpu.VMEM((_C,), jnp.int32),
            pltpu.VMEM((_C, _HH), jnp.float32),
            pltpu.VMEM((_C, _HH), jnp.float32),
            pltpu.SemaphoreType.DMA,
            pltpu.SemaphoreType.DMA,
        ],
    )
    def sc_layer(a_hbm, b_hbm, src_hbm, dst_hbm, z_hbm, out_hbm,
                 acc, src_v, dst_v, idxa_v, idxb_v, abuf, bbuf, sem_a, sem_b):
        c = lax.axis_index("c")
        s = lax.axis_index("s")
        row0 = s * rpt
        coff = c * _N
        ooff = c * _NP
        pltpu.sync_copy(z_hbm, acc.at[pl.ds(row0, rpt)])
        plsc.subcore_barrier()

        @pl.loop(0, chunks)
        def _chunk(g):
            eoff = s * ept + g * _C
            pltpu.sync_copy(src_hbm.at[pl.ds(eoff, _C)], src_v)
            pltpu.sync_copy(dst_hbm.at[pl.ds(eoff, _C)], dst_v)
            for k in range(_C // 16):
                sl = pl.ds(k * 16, 16)
                idxa_v[sl] = src_v[sl] + coff
                idxb_v[sl] = dst_v[sl] + coff
            ca = pltpu.async_copy(a_hbm.at[idxa_v], abuf, sem_a)
            cb = pltpu.async_copy(b_hbm.at[idxb_v], bbuf, sem_b)
            ca.wait()
            cb.wait()

            @pl.loop(0, _C)
            def _row(r):
                for k in range(_HH // 16):
                    sl = pl.ds(k * 16, 16)
                    abuf[r, sl] = jnp.maximum(abuf[r, sl] + bbuf[r, sl], 0.0)

            pltpu.sync_copy(abuf, acc.at[dst_v], add=True)

        plsc.subcore_barrier()
        pltpu.sync_copy(acc.at[pl.ds(row0, rpt)],
                        out_hbm.at[pl.ds(ooff + row0, rpt)])

    return sc_layer


def kernel(node_features, edge_index, W_emb, b_emb, W_g0, b_g0,
           W_g1, b_g1, W_g2, b_g2, W_p1, b_p1, W_p2, b_p2):
    E = edge_index.shape[1]
    src = edge_index[0]
    dst = edge_index[1]
    zeros = jnp.zeros((_NP // _NS, _HH), jnp.float32)
    sc_layer = _make_sc_layer(E)

    h, a3, b3 = _mm_embed(node_features, W_emb, b_emb.reshape(1, _H),
                          W_g0, b_g0.reshape(1, _H))
    m = sc_layer(a3.reshape(_NC * _N, _HH), b3.reshape(_NC * _N, _HH),
                 src, dst, zeros)
    h, a3, b3 = _mm_combine(h, m.reshape(_NC, _NP, _HH),
                            W_g1, b_g1.reshape(1, _H))
    m = sc_layer(a3.reshape(_NC * _N, _HH), b3.reshape(_NC * _N, _HH),
                 src, dst, zeros)
    h, a3, b3 = _mm_combine(h, m.reshape(_NC, _NP, _HH),
                            W_g2, b_g2.reshape(1, _H))
    m = sc_layer(a3.reshape(_NC * _N, _HH), b3.reshape(_NC * _N, _HH),
                 src, dst, zeros)

    wp2 = jnp.pad(W_p2, ((0, 0), (0, _HH - 1)))
    bp2 = jnp.pad(b_p2, (0, _HH - 1)).reshape(1, _HH)
    hf, pred = _mm_final(h, m.reshape(_NC, _NP, _HH),
                         W_p1, b_p1.reshape(1, _H), wp2, bp2)
    return (pred[0, :1], hf)
